# Initial kernel scaffold; baseline (speedup 1.0000x reference)
import functools

import jax
import jax.numpy as jnp
from jax import lax
from jax.experimental import pallas as pl
from jax.experimental.pallas import tpu as pltpu
from jax.experimental.pallas import tpu_sc as plsc

_BATCH = 16384
_HIST = 50
_EMBED = 64
_N = _BATCH * _HIST
_NC = 2
_NS = 16
_NW = _NC * _NS
_PER_W = _N // _NW
_SUB = 128
_NSUB = 4
_CH = _SUB * _NSUB
_NCH = _PER_W // _CH

_mesh = plsc.VectorSubcoreMesh(core_axis_name="c", subcore_axis_name="s")


@functools.partial(
    pl.kernel,
    mesh=_mesh,
    out_type=jax.ShapeDtypeStruct((_N, _EMBED), jnp.float32),
    scratch_types=[
        pltpu.VMEM((2, _NSUB, _SUB), jnp.int32),
        pltpu.VMEM((2, _CH, _EMBED), jnp.float32),
        pltpu.SemaphoreType.DMA,
        pltpu.SemaphoreType.DMA,
    ],
)
def _emb_lookup(table_hbm, idx_hbm, out_hbm, idx_v, rows_v, sem0, sem1):
    sems = (sem0, sem1)
    wid = lax.axis_index("s") * _NC + lax.axis_index("c")
    base = wid * _PER_W
    row_base = wid * (_PER_W // _SUB)

    def start_chunk(chunk, b):
        pltpu.sync_copy(
            idx_hbm.at[pl.ds(row_base + chunk * _NSUB, _NSUB)], idx_v.at[b]
        )
        for j in range(_NSUB):
            pltpu.async_copy(
                table_hbm.at[idx_v.at[b, j]],
                rows_v.at[b, pl.ds(j * _SUB, _SUB)],
                sems[b],
            )

    def finish_chunk(chunk, b):
        pltpu.make_async_copy(
            out_hbm.at[pl.ds(0, _CH)], rows_v.at[b], sems[b]
        ).wait()
        pltpu.sync_copy(rows_v.at[b], out_hbm.at[pl.ds(base + chunk * _CH, _CH)])

    start_chunk(0, 0)

    def loop_body(g, carry):
        c0 = g * 2
        start_chunk(c0 + 1, 1)
        finish_chunk(c0, 0)

        @pl.when(c0 + 2 < _NCH)
        def _():
            start_chunk(c0 + 2, 0)

        finish_chunk(c0 + 1, 1)
        return carry

    lax.fori_loop(0, _NCH // 2, loop_body, 0)


def kernel(x, table):
    idx = x.reshape(-1).astype(jnp.int32).reshape(_N // _SUB, _SUB)
    out = _emb_lookup(table, idx)
    return out.reshape(_BATCH, _HIST, _EMBED)


# speedup vs baseline: 1.8538x; 1.8538x over previous
import functools

import jax
import jax.numpy as jnp
from jax import lax
from jax.experimental import pallas as pl
from jax.experimental.pallas import tpu as pltpu
from jax.experimental.pallas import tpu_sc as plsc

_BATCH = 16384
_HIST = 50
_EMBED = 64
_N = _BATCH * _HIST
_NC = 2
_NS = 16
_NW = _NC * _NS
_PER_W = _N // _NW
_SUB = 128
_NSUB = 4
_CH = _SUB * _NSUB
_NCH = _PER_W // _CH

_mesh = plsc.VectorSubcoreMesh(core_axis_name="c", subcore_axis_name="s")


@functools.partial(
    pl.kernel,
    mesh=_mesh,
    out_type=jax.ShapeDtypeStruct((_N, _EMBED), jnp.float32),
    scratch_types=[
        pltpu.VMEM((2, _NSUB, _SUB), jnp.int32),
        pltpu.VMEM((2, _CH, _EMBED), jnp.float32),
        pltpu.SemaphoreType.DMA,
        pltpu.SemaphoreType.DMA,
    ],
    compiler_params=pltpu.CompilerParams(use_tc_tiling_on_sc=False),
)
def _emb_lookup(table_hbm, idx_hbm, out_hbm, idx_v, rows_v, sem0, sem1):
    sems = (sem0, sem1)
    wid = lax.axis_index("s") * _NC + lax.axis_index("c")
    base = wid * _PER_W
    row_base = wid * (_PER_W // _SUB)

    def start_chunk(chunk, b):
        pltpu.sync_copy(
            idx_hbm.at[pl.ds(row_base + chunk * _NSUB, _NSUB)], idx_v.at[b]
        )
        for j in range(_NSUB):
            pltpu.async_copy(
                table_hbm.at[idx_v.at[b, j]],
                rows_v.at[b, pl.ds(j * _SUB, _SUB)],
                sems[b],
            )

    def finish_chunk(chunk, b):
        pltpu.make_async_copy(
            out_hbm.at[pl.ds(0, _CH)], rows_v.at[b], sems[b]
        ).wait()
        pltpu.sync_copy(rows_v.at[b], out_hbm.at[pl.ds(base + chunk * _CH, _CH)])

    start_chunk(0, 0)

    def loop_body(g, carry):
        c0 = g * 2
        start_chunk(c0 + 1, 1)
        finish_chunk(c0, 0)

        @pl.when(c0 + 2 < _NCH)
        def _():
            start_chunk(c0 + 2, 0)

        finish_chunk(c0 + 1, 1)
        return carry

    lax.fori_loop(0, _NCH // 2, loop_body, 0)


def kernel(x, table):
    idx = x.reshape(-1).astype(jnp.int32).reshape(_N // _SUB, _SUB)
    out = _emb_lookup(table, idx)
    return out.reshape(_BATCH, _HIST, _EMBED)


# speedup vs baseline: 1.8749x; 1.0114x over previous
import functools

import jax
import jax.numpy as jnp
from jax import lax
from jax.experimental import pallas as pl
from jax.experimental.pallas import tpu as pltpu
from jax.experimental.pallas import tpu_sc as plsc

_BATCH = 16384
_HIST = 50
_EMBED = 64
_N = _BATCH * _HIST
_NC = 2
_NS = 16
_NW = _NC * _NS
_PER_W = _N // _NW
_CHUNK = 128
_NCH = _PER_W // _CHUNK
_NBUF = 8
_LOOK = 4
_NGRP = _NCH // _NBUF

_mesh = plsc.VectorSubcoreMesh(core_axis_name="c", subcore_axis_name="s")


@functools.partial(
    pl.kernel,
    mesh=_mesh,
    out_type=jax.ShapeDtypeStruct((_N, _EMBED), jnp.float32),
    scratch_types=[
        pltpu.VMEM((_NCH, _CHUNK), jnp.int32),
        pltpu.VMEM((_NBUF, _CHUNK, _EMBED), jnp.float32),
        pltpu.SemaphoreType.DMA((_NBUF,)),
        pltpu.SemaphoreType.DMA((_NBUF,)),
    ],
    compiler_params=pltpu.CompilerParams(use_tc_tiling_on_sc=False),
)
def _emb_lookup(table_hbm, idx_hbm, out_hbm, idx_v, rows_v, gsem, ssem):
    wid = lax.axis_index("s") * _NC + lax.axis_index("c")
    base = wid * _PER_W

    pltpu.sync_copy(idx_hbm.at[pl.ds(wid * _NCH, _NCH)], idx_v)

    def start_gather(chunk, b):
        pltpu.async_copy(table_hbm.at[idx_v.at[chunk]], rows_v.at[b], gsem.at[b])

    def wait_gather(b):
        pltpu.make_async_copy(
            out_hbm.at[pl.ds(0, _CHUNK)], rows_v.at[b], gsem.at[b]
        ).wait()

    def start_store(chunk, b):
        pltpu.async_copy(
            rows_v.at[b], out_hbm.at[pl.ds(base + chunk * _CHUNK, _CHUNK)], ssem.at[b]
        )

    def wait_store(b):
        pltpu.make_async_copy(
            rows_v.at[b], out_hbm.at[pl.ds(0, _CHUNK)], ssem.at[b]
        ).wait()

    for b in range(_LOOK):
        start_gather(b, b)

    def loop_body(g, carry):
        c0 = g * _NBUF
        for b in range(_NBUF):
            c = c0 + b
            cn = c + _LOOK
            bn = (b + _LOOK) % _NBUF

            @pl.when(cn >= _NBUF)
            def _():
                wait_store(bn)

            @pl.when(cn < _NCH)
            def _():
                start_gather(cn, bn)

            wait_gather(b)
            start_store(c, b)
        return carry

    lax.fori_loop(0, _NGRP, loop_body, 0)

    for i in range(_LOOK):
        wait_store((_LOOK + i) % _NBUF)


def kernel(x, table):
    idx = x.reshape(-1).astype(jnp.int32).reshape(_N // _CHUNK, _CHUNK)
    out = _emb_lookup(table, idx)
    return out.reshape(_BATCH, _HIST, _EMBED)
